# Initial kernel scaffold; baseline (speedup 1.0000x reference)
import jax
import jax.numpy as jnp
from jax.experimental import pallas as pl


def kernel(x, edge_index, W1, b1, W2, b2, W3, b3):
    raise NotImplementedError("write your pallas kernel here")


# speedup vs baseline: 4.8959x; 4.8959x over previous
import functools

import jax
import jax.numpy as jnp
from jax import lax
from jax.experimental import pallas as pl
from jax.experimental.pallas import tpu as pltpu
import jax.experimental.pallas.tpu_sc as plsc

N = 10000
E = 320000
NC, NS, L = 2, 16, 16
NW = NC * NS
NPAD = 10240
CPW = NPAD // NW
R = 400


def _sc_degree(dst):
    epw = E // NW
    cpt = NPAD // NS
    mesh = plsc.VectorSubcoreMesh(core_axis_name="c", subcore_axis_name="s")

    @functools.partial(
        pl.kernel,
        out_type=jax.ShapeDtypeStruct((2 * NPAD,), jnp.float32),
        mesh=mesh,
        scratch_types=[
            pltpu.VMEM((epw,), jnp.int32),
            pltpu.VMEM((NPAD,), jnp.float32),
            pltpu.VMEM((NS * cpt,), jnp.float32),
            pltpu.VMEM((cpt,), jnp.float32),
            pltpu.VMEM_SHARED((NS * NPAD,), jnp.float32),
        ],
        compiler_params=pltpu.CompilerParams(needs_layout_passes=False),
    )
    def k(dst_hbm, out_hbm, dst_v, hist_v, red_v, res_v, shared):
        c = lax.axis_index("c")
        s = lax.axis_index("s")
        wid = s * NC + c

        pltpu.sync_copy(dst_hbm.at[pl.ds(wid * epw, epw)], dst_v)

        def zero_body(j, _):
            hist_v[pl.ds(j * L, L)] = jnp.zeros((L,), jnp.float32)
            return 0
        lax.fori_loop(0, NPAD // L, zero_body, 0)

        ones = jnp.full((L,), 1.0, jnp.float32)

        def count_body(j, _):
            idx = dst_v[pl.ds(j * L, L)]
            plsc.addupdate_scatter(hist_v, [idx], ones)
            return 0
        lax.fori_loop(0, epw // L, count_body, 0)

        pltpu.sync_copy(hist_v, shared.at[pl.ds(s * NPAD, NPAD)])
        plsc.subcore_barrier()

        for t in range(NS):
            pltpu.sync_copy(shared.at[pl.ds(t * NPAD + s * cpt, cpt)],
                            red_v.at[pl.ds(t * cpt, cpt)])

        def red_body(j, _):
            acc = jnp.zeros((L,), jnp.float32)

            def add_t(t, a):
                return a + red_v[pl.ds(t * cpt + j * L, L)]
            acc = lax.fori_loop(0, NS, add_t, acc)
            res_v[pl.ds(j * L, L)] = acc
            return 0
        lax.fori_loop(0, cpt // L, red_body, 0)

        pltpu.sync_copy(res_v, out_hbm.at[pl.ds(c * NPAD + s * cpt, cpt)])

    return k(dst)


def _sc_propagate(y2, src, dst, fh):
    ept = E // NS
    nch = ept // L
    rpt = NPAD // NS
    mesh = plsc.VectorSubcoreMesh(core_axis_name="c", subcore_axis_name="s")

    @functools.partial(
        pl.kernel,
        out_type=jax.ShapeDtypeStruct((2 * NPAD, fh), jnp.float32),
        mesh=mesh,
        scratch_types=[
            pltpu.VMEM((ept,), jnp.int32),
            pltpu.VMEM((ept,), jnp.int32),
            pltpu.VMEM((L, fh), jnp.float32),
            pltpu.VMEM_SHARED((NPAD, fh), jnp.float32),
            pltpu.SemaphoreType.DMA,
        ],
        compiler_params=pltpu.CompilerParams(needs_layout_passes=False,
                                             use_tc_tiling_on_sc=False),
    )
    def k(y_hbm, src_hbm, dst_hbm, out_hbm, src_v, dst_v, gbuf, accum, sem):
        c = lax.axis_index("c")
        s = lax.axis_index("s")
        cn = c * NPAD

        pltpu.sync_copy(src_hbm.at[pl.ds(s * ept, ept)], src_v)
        pltpu.sync_copy(dst_hbm.at[pl.ds(s * ept, ept)], dst_v)

        pltpu.sync_copy(y_hbm.at[pl.ds(cn + s * rpt, rpt)],
                        accum.at[pl.ds(s * rpt, rpt)])
        plsc.subcore_barrier()

        def body(j, _):
            sidx = src_v[pl.ds(j * L, L)] + cn
            didx = dst_v[pl.ds(j * L, L)]
            pltpu.async_copy(y_hbm.at[sidx], gbuf, sem).wait()
            pltpu.sync_copy(gbuf, accum.at[didx], add=True)
            return 0
        lax.fori_loop(0, nch, body, 0)

        plsc.subcore_barrier()
        pltpu.sync_copy(accum.at[pl.ds(s * rpt, rpt)],
                        out_hbm.at[pl.ds(cn + s * rpt, rpt)])

    return k(y2, src, dst)


def _grid_spec(in_specs, out_block):
    return dict(
        grid=(N // R,),
        in_specs=in_specs,
        out_specs=pl.BlockSpec(out_block, lambda i: (0, i, 0)),
    )


def _full(shape):
    nd = len(shape)
    return pl.BlockSpec(shape, lambda i: (0,) * nd)


def _tc_layer1(x, W, deg2):
    fo = W.shape[1]

    def body(x_ref, w_ref, d_ref, o_ref):
        dinv = lax.rsqrt(d_ref[...] + 1.0)
        y = jnp.dot(x_ref[...], w_ref[...],
                    preferred_element_type=jnp.float32) * dinv
        o_ref[0] = y[:, : fo // 2]
        o_ref[1] = y[:, fo // 2:]

    return pl.pallas_call(
        body,
        out_shape=jax.ShapeDtypeStruct((2, NPAD, fo // 2), jnp.float32),
        **_grid_spec(
            [pl.BlockSpec((R, x.shape[1]), lambda i: (i, 0)),
             _full(W.shape),
             pl.BlockSpec((R, 1), lambda i: (i, 0))],
            (2, R, fo // 2),
        ),
    )(x, W, deg2)


def _tc_layer(z, b, W, deg2):
    fo = W.shape[1]

    def body(z_ref, b_ref, w_ref, d_ref, o_ref):
        dinv = lax.rsqrt(d_ref[...] + 1.0)
        zc = jnp.concatenate([z_ref[0], z_ref[1]], axis=1)
        h = jnp.maximum(zc * dinv + b_ref[...], 0.0)
        y = jnp.dot(h, w_ref[...], preferred_element_type=jnp.float32) * dinv
        o_ref[0] = y[:, : fo // 2]
        o_ref[1] = y[:, fo // 2:]

    fh = z.shape[2]
    return pl.pallas_call(
        body,
        out_shape=jax.ShapeDtypeStruct((2, NPAD, fo // 2), jnp.float32),
        **_grid_spec(
            [pl.BlockSpec((2, R, fh), lambda i: (0, i, 0)),
             _full(b.shape),
             _full(W.shape),
             pl.BlockSpec((R, 1), lambda i: (i, 0))],
            (2, R, fo // 2),
        ),
    )(z, b, W, deg2)


def _tc_final(z, b, deg2):
    fh = z.shape[2]

    def body(z_ref, b_ref, d_ref, o_ref):
        dinv = lax.rsqrt(d_ref[...] + 1.0)
        zc = jnp.concatenate([z_ref[0], z_ref[1]], axis=1)
        o_ref[...] = jax.nn.sigmoid(zc * dinv + b_ref[...])

    return pl.pallas_call(
        body,
        out_shape=jax.ShapeDtypeStruct((N, 2 * fh), jnp.float32),
        grid=(N // R,),
        in_specs=[pl.BlockSpec((2, R, fh), lambda i: (0, i, 0)),
                  _full(b.shape),
                  pl.BlockSpec((R, 1), lambda i: (i, 0))],
        out_specs=pl.BlockSpec((R, 2 * fh), lambda i: (i, 0)),
    )(z, b, deg2)


def kernel(x, edge_index, W1, b1, W2, b2, W3, b3):
    src = edge_index[0].astype(jnp.int32)
    dst = edge_index[1].astype(jnp.int32)

    degf = _sc_degree(dst)
    deg2 = (degf[:N] + degf[NPAD:NPAD + N]).reshape(N, 1)

    y1 = _tc_layer1(x, W1, deg2).reshape(2 * NPAD, 128)
    z1 = _sc_propagate(y1, src, dst, 128).reshape(2, NPAD, 128)

    y2 = _tc_layer(z1, b1.reshape(1, -1), W2, deg2).reshape(2 * NPAD, 128)
    z2 = _sc_propagate(y2, src, dst, 128).reshape(2, NPAD, 128)

    y3 = _tc_layer(z2, b2.reshape(1, -1), W3, deg2).reshape(2 * NPAD, 32)
    z3 = _sc_propagate(y3, src, dst, 32).reshape(2, NPAD, 32)

    return _tc_final(z3, b3.reshape(1, -1), deg2)


# speedup vs baseline: 16.7255x; 3.4163x over previous
import functools

import jax
import jax.numpy as jnp
from jax import lax
from jax.experimental import pallas as pl
from jax.experimental.pallas import tpu as pltpu
import jax.experimental.pallas.tpu_sc as plsc

N = 10000
E = 320000
NC, NS, L = 2, 16, 16
NW = NC * NS
NPAD = 10240
R = 400
CH = 32
NBUF = 5
G = 2


def _sc_degree(dst):
    epw = E // NW
    cpt = NPAD // NS
    mesh = plsc.VectorSubcoreMesh(core_axis_name="c", subcore_axis_name="s")

    @functools.partial(
        pl.kernel,
        out_type=jax.ShapeDtypeStruct((2 * NPAD,), jnp.float32),
        mesh=mesh,
        scratch_types=[
            pltpu.VMEM((epw,), jnp.int32),
            pltpu.VMEM((NPAD,), jnp.float32),
            pltpu.VMEM((NS * cpt,), jnp.float32),
            pltpu.VMEM((cpt,), jnp.float32),
            pltpu.VMEM_SHARED((NS * NPAD,), jnp.float32),
        ],
        compiler_params=pltpu.CompilerParams(needs_layout_passes=False),
    )
    def k(dst_hbm, out_hbm, dst_v, hist_v, red_v, res_v, shared):
        c = lax.axis_index("c")
        s = lax.axis_index("s")
        wid = s * NC + c

        pltpu.sync_copy(dst_hbm.at[pl.ds(wid * epw, epw)], dst_v)

        def zero_body(j, _):
            hist_v[pl.ds(j * L, L)] = jnp.zeros((L,), jnp.float32)
            return 0
        lax.fori_loop(0, NPAD // L, zero_body, 0)

        ones = jnp.full((L,), 1.0, jnp.float32)

        def count_body(j, _):
            idx = dst_v[pl.ds(j * L, L)]
            plsc.addupdate_scatter(hist_v, [idx], ones)
            return 0
        lax.fori_loop(0, epw // L, count_body, 0)

        pltpu.sync_copy(hist_v, shared.at[pl.ds(s * NPAD, NPAD)])
        plsc.subcore_barrier()

        for t in range(NS):
            pltpu.sync_copy(shared.at[pl.ds(t * NPAD + s * cpt, cpt)],
                            red_v.at[pl.ds(t * cpt, cpt)])

        def red_body(j, _):
            acc = jnp.zeros((L,), jnp.float32)

            def add_t(t, a):
                return a + red_v[pl.ds(t * cpt + j * L, L)]
            acc = lax.fori_loop(0, NS, add_t, acc)
            res_v[pl.ds(j * L, L)] = acc
            return 0
        lax.fori_loop(0, cpt // L, red_body, 0)

        pltpu.sync_copy(res_v, out_hbm.at[pl.ds(c * NPAD + s * cpt, cpt)])

    return k(dst)


def _sc_propagate(y0, y1, ecnk, fh):
    ept = E // NS
    nch = ept // CH
    ngrp = nch // NBUF
    rpt = NPAD // NS
    mesh = plsc.VectorSubcoreMesh(core_axis_name="c", subcore_axis_name="s")
    half = jax.ShapeDtypeStruct((NPAD, fh), jnp.float32)

    @functools.partial(
        pl.kernel,
        out_type=(half, half),
        mesh=mesh,
        scratch_types=[
            pltpu.VMEM((NBUF, 2, CH), jnp.int32),
            pltpu.VMEM((NBUF, CH, fh), jnp.float32),
            pltpu.VMEM_SHARED((NPAD, fh), jnp.float32),
            [pltpu.SemaphoreType.DMA] * NBUF,
            [pltpu.SemaphoreType.DMA] * NBUF,
        ],
        compiler_params=pltpu.CompilerParams(needs_layout_passes=False,
                                             use_tc_tiling_on_sc=False),
    )
    def k(y0_hbm, y1_hbm, e_hbm, z0_hbm, z1_hbm,
          ibuf, gbuf, accum, isems, gsems):
        c = lax.axis_index("c")
        s = lax.axis_index("s")
        base = s * nch

        rlo = s * rpt

        @pl.when(c == 0)
        def _():
            pltpu.sync_copy(y0_hbm.at[pl.ds(rlo, rpt)],
                            accum.at[pl.ds(rlo, rpt)])

        @pl.when(c == 1)
        def _():
            pltpu.sync_copy(y1_hbm.at[pl.ds(rlo, rpt)],
                            accum.at[pl.ds(rlo, rpt)])

        def fire_idx(j, slot):
            pltpu.async_copy(e_hbm.at[base + j], ibuf.at[slot], isems[slot])

        def wait_idx(slot):
            pltpu.make_async_copy(e_hbm.at[0], ibuf.at[slot],
                                  isems[slot]).wait()

        def fire_gather(j, slot):
            idx = ibuf.at[slot, 0]

            @pl.when(c == 0)
            def _():
                pltpu.async_copy(y0_hbm.at[idx], gbuf.at[slot], gsems[slot])

            @pl.when(c == 1)
            def _():
                pltpu.async_copy(y1_hbm.at[idx], gbuf.at[slot], gsems[slot])

        def wait_gather(slot):
            pltpu.make_async_copy(y0_hbm.at[pl.ds(0, CH)], gbuf.at[slot],
                                  gsems[slot]).wait()

        plsc.subcore_barrier()

        for b in range(NBUF):
            fire_idx(b, b)
        for b in range(G):
            wait_idx(b)
            fire_gather(b, b)

        def outer(g, _):
            for b in range(NBUF):
                j = g * NBUF + b
                gslot = (b + G) % NBUF

                @pl.when(j < nch - G)
                def _():
                    wait_idx(gslot)
                    fire_gather(j + G, gslot)

                wait_gather(b)
                for t in range(CH // L):
                    didx = ibuf[b, 1, pl.ds(t * L, L)]
                    pltpu.sync_copy(gbuf.at[b, pl.ds(t * L, L)],
                                    accum.at[didx], add=True)

                @pl.when(j < nch - NBUF)
                def _():
                    fire_idx(j + NBUF, b)
            return 0
        lax.fori_loop(0, ngrp, outer, 0)

        plsc.subcore_barrier()

        @pl.when(c == 0)
        def _():
            pltpu.sync_copy(accum.at[pl.ds(rlo, rpt)],
                            z0_hbm.at[pl.ds(rlo, rpt)])

        @pl.when(c == 1)
        def _():
            pltpu.sync_copy(accum.at[pl.ds(rlo, rpt)],
                            z1_hbm.at[pl.ds(rlo, rpt)])

    return k(y0, y1, ecnk)


def _full(shape):
    nd = len(shape)
    return pl.BlockSpec(shape, lambda i: (0,) * nd)


def _row(block):
    return pl.BlockSpec(block, lambda i: (i,) + (0,) * (len(block) - 1))


def _halves_out(fo):
    h = jax.ShapeDtypeStruct((NPAD, fo // 2), jnp.float32)
    return dict(
        out_shape=(h, h),
        out_specs=(_row((R, fo // 2)), _row((R, fo // 2))),
    )


def _tc_layer1(x, W, deg2):
    fo = W.shape[1]

    def body(x_ref, w_ref, d_ref, o0_ref, o1_ref):
        dinv = lax.rsqrt(d_ref[...] + 1.0)
        y = jnp.dot(x_ref[...], w_ref[...],
                    preferred_element_type=jnp.float32) * dinv
        o0_ref[...] = y[:, : fo // 2]
        o1_ref[...] = y[:, fo // 2:]

    return pl.pallas_call(
        body,
        grid=(N // R,),
        in_specs=[_row((R, x.shape[1])), _full(W.shape), _row((R, 1))],
        **_halves_out(fo),
    )(x, W, deg2)


def _tc_layer(z0, z1, b, W, deg2):
    fo = W.shape[1]
    fh = z0.shape[1]

    def body(z0_ref, z1_ref, b_ref, w_ref, d_ref, o0_ref, o1_ref):
        dinv = lax.rsqrt(d_ref[...] + 1.0)
        zc = jnp.concatenate([z0_ref[...], z1_ref[...]], axis=1)
        h = jnp.maximum(zc * dinv + b_ref[...], 0.0)
        y = jnp.dot(h, w_ref[...], preferred_element_type=jnp.float32) * dinv
        o0_ref[...] = y[:, : fo // 2]
        o1_ref[...] = y[:, fo // 2:]

    return pl.pallas_call(
        body,
        grid=(N // R,),
        in_specs=[_row((R, fh)), _row((R, fh)), _full(b.shape),
                  _full(W.shape), _row((R, 1))],
        **_halves_out(fo),
    )(z0, z1, b, W, deg2)


def _tc_final(z0, z1, b, deg2):
    fh = z0.shape[1]

    def body(z0_ref, z1_ref, b_ref, d_ref, o_ref):
        dinv = lax.rsqrt(d_ref[...] + 1.0)
        zc = jnp.concatenate([z0_ref[...], z1_ref[...]], axis=1)
        o_ref[...] = jax.nn.sigmoid(zc * dinv + b_ref[...])

    return pl.pallas_call(
        body,
        grid=(N // R,),
        in_specs=[_row((R, fh)), _row((R, fh)), _full(b.shape), _row((R, 1))],
        out_shape=jax.ShapeDtypeStruct((N, 2 * fh), jnp.float32),
        out_specs=_row((R, 2 * fh)),
    )(z0, z1, b, deg2)


def kernel(x, edge_index, W1, b1, W2, b2, W3, b3):
    src = edge_index[0].astype(jnp.int32)
    dst = edge_index[1].astype(jnp.int32)
    ecnk = jnp.stack([src.reshape(-1, CH), dst.reshape(-1, CH)], axis=1)

    degf = _sc_degree(dst)
    deg2 = (degf[:N] + degf[NPAD:NPAD + N]).reshape(N, 1)

    y0, y1 = _tc_layer1(x, W1, deg2)
    z0, z1 = _sc_propagate(y0, y1, ecnk, 128)

    y0, y1 = _tc_layer(z0, z1, b1.reshape(1, -1), W2, deg2)
    z0, z1 = _sc_propagate(y0, y1, ecnk, 128)

    y0, y1 = _tc_layer(z0, z1, b2.reshape(1, -1), W3, deg2)
    z0, z1 = _sc_propagate(y0, y1, ecnk, 32)

    return _tc_final(z0, z1, b3.reshape(1, -1), deg2)


# speedup vs baseline: 23.4492x; 1.4020x over previous
import functools

import jax
import jax.numpy as jnp
from jax import lax
from jax.experimental import pallas as pl
from jax.experimental.pallas import tpu as pltpu
import jax.experimental.pallas.tpu_sc as plsc

N = 10000
E = 320000
NC, NS, L = 2, 16, 16
NW = NC * NS
NPAD = 10240
R = 400
CH = 80
NBUF = 4
G = 2


def _sc_degree(dst):
    epw = E // NW
    cpt = NPAD // NS
    mesh = plsc.VectorSubcoreMesh(core_axis_name="c", subcore_axis_name="s")

    @functools.partial(
        pl.kernel,
        out_type=jax.ShapeDtypeStruct((2 * NPAD,), jnp.float32),
        mesh=mesh,
        scratch_types=[
            pltpu.VMEM((epw,), jnp.int32),
            pltpu.VMEM((NPAD,), jnp.float32),
            pltpu.VMEM((NS * cpt,), jnp.float32),
            pltpu.VMEM((cpt,), jnp.float32),
            pltpu.VMEM_SHARED((NS * NPAD,), jnp.float32),
        ],
        compiler_params=pltpu.CompilerParams(needs_layout_passes=False),
    )
    def k(dst_hbm, out_hbm, dst_v, hist_v, red_v, res_v, shared):
        c = lax.axis_index("c")
        s = lax.axis_index("s")
        wid = s * NC + c

        pltpu.sync_copy(dst_hbm.at[pl.ds(wid * epw, epw)], dst_v)

        def zero_body(j, _):
            hist_v[pl.ds(j * L, L)] = jnp.zeros((L,), jnp.float32)
            return 0
        lax.fori_loop(0, NPAD // L, zero_body, 0)

        ones = jnp.full((L,), 1.0, jnp.float32)

        def count_body(j, _):
            idx = dst_v[pl.ds(j * L, L)]
            plsc.addupdate_scatter(hist_v, [idx], ones)
            return 0
        lax.fori_loop(0, epw // L, count_body, 0)

        pltpu.sync_copy(hist_v, shared.at[pl.ds(s * NPAD, NPAD)])
        plsc.subcore_barrier()

        for t in range(NS):
            pltpu.sync_copy(shared.at[pl.ds(t * NPAD + s * cpt, cpt)],
                            red_v.at[pl.ds(t * cpt, cpt)])

        def red_body(j, _):
            acc = jnp.zeros((L,), jnp.float32)

            def add_t(t, a):
                return a + red_v[pl.ds(t * cpt + j * L, L)]
            acc = lax.fori_loop(0, NS, add_t, acc)
            res_v[pl.ds(j * L, L)] = acc
            return 0
        lax.fori_loop(0, cpt // L, red_body, 0)

        pltpu.sync_copy(res_v, out_hbm.at[pl.ds(c * NPAD + s * cpt, cpt)])

    return k(dst)


def _sc_propagate(y0, y1, ecnk, fh):
    ept = E // NS
    nch = ept // CH
    NI = 2 * NBUF
    ngrp2 = nch // NI
    tail = nch - ngrp2 * NI
    rpt = NPAD // NS
    mesh = plsc.VectorSubcoreMesh(core_axis_name="c", subcore_axis_name="s")
    half = jax.ShapeDtypeStruct((NPAD, fh), jnp.float32)

    @functools.partial(
        pl.kernel,
        out_type=(half, half),
        mesh=mesh,
        scratch_types=[
            pltpu.VMEM((NI, 2, CH), jnp.int32),
            pltpu.VMEM((NBUF, CH, fh), jnp.float32),
            pltpu.VMEM_SHARED((NPAD, fh), jnp.float32),
            [pltpu.SemaphoreType.DMA] * NI,
            [pltpu.SemaphoreType.DMA] * NBUF,
            [pltpu.SemaphoreType.DMA] * NBUF,
        ],
        compiler_params=pltpu.CompilerParams(needs_layout_passes=False,
                                             use_tc_tiling_on_sc=False),
    )
    def k(y0_hbm, y1_hbm, e_hbm, z0_hbm, z1_hbm,
          ibuf, gbuf, accum, isems, gsems, ssems):
        c = lax.axis_index("c")
        s = lax.axis_index("s")
        base = s * nch

        rlo = s * rpt

        @pl.when(c == 0)
        def _():
            pltpu.sync_copy(y0_hbm.at[pl.ds(rlo, rpt)],
                            accum.at[pl.ds(rlo, rpt)])

        @pl.when(c == 1)
        def _():
            pltpu.sync_copy(y1_hbm.at[pl.ds(rlo, rpt)],
                            accum.at[pl.ds(rlo, rpt)])

        def fire_idx(j, islot):
            pltpu.async_copy(e_hbm.at[base + j], ibuf.at[islot], isems[islot])

        def wait_idx(islot):
            pltpu.make_async_copy(e_hbm.at[0], ibuf.at[islot],
                                  isems[islot]).wait()

        def fire_gather(islot, slot):
            idx = ibuf.at[islot, 0]

            @pl.when(c == 0)
            def _():
                pltpu.async_copy(y0_hbm.at[idx], gbuf.at[slot], gsems[slot])

            @pl.when(c == 1)
            def _():
                pltpu.async_copy(y1_hbm.at[idx], gbuf.at[slot], gsems[slot])

        def wait_gather(slot):
            pltpu.make_async_copy(y0_hbm.at[pl.ds(0, CH)], gbuf.at[slot],
                                  gsems[slot]).wait()

        def fire_scatter(islot, slot):
            pltpu.async_copy(gbuf.at[slot], accum.at[ibuf.at[islot, 1]],
                             ssems[slot], add=True)

        def drain_scatter(slot):
            pltpu.make_async_copy(gbuf.at[slot], accum.at[ibuf.at[0, 1]],
                                  ssems[slot]).wait()

        plsc.subcore_barrier()

        for b in range(NBUF):
            fire_idx(b, b)
        for b in range(G):
            wait_idx(b)
            fire_gather(b, b)

        def chunk_step(j, b, islot):
            gslot = (b + G) % NBUF
            gislot = (islot + G) % NI

            @pl.when(j < nch - G)
            def _():
                @pl.when(j + G >= NBUF)
                def _():
                    drain_scatter(gslot)
                wait_idx(gislot)
                fire_gather(gislot, gslot)

            wait_gather(b)
            fire_scatter(islot, b)

            @pl.when(j < nch - NBUF)
            def _():
                fire_idx(j + NBUF, (islot + NBUF) % NI)

        def outer(g2, _):
            for p in range(NI):
                j = g2 * NI + p
                chunk_step(j, p % NBUF, p)
            return 0
        lax.fori_loop(0, ngrp2, outer, 0)

        for p in range(tail):
            j = ngrp2 * NI + p
            chunk_step(jnp.int32(j), p % NBUF, p)

        for b in range(NBUF):
            drain_scatter(b)

        plsc.subcore_barrier()

        @pl.when(c == 0)
        def _():
            pltpu.sync_copy(accum.at[pl.ds(rlo, rpt)],
                            z0_hbm.at[pl.ds(rlo, rpt)])

        @pl.when(c == 1)
        def _():
            pltpu.sync_copy(accum.at[pl.ds(rlo, rpt)],
                            z1_hbm.at[pl.ds(rlo, rpt)])

    return k(y0, y1, ecnk)


def _full(shape):
    nd = len(shape)
    return pl.BlockSpec(shape, lambda i: (0,) * nd)


def _row(block):
    return pl.BlockSpec(block, lambda i: (i,) + (0,) * (len(block) - 1))


def _halves_out(fo):
    h = jax.ShapeDtypeStruct((NPAD, fo // 2), jnp.float32)
    return dict(
        out_shape=(h, h),
        out_specs=(_row((R, fo // 2)), _row((R, fo // 2))),
    )


def _tc_layer1(x, W, deg2):
    fo = W.shape[1]

    def body(x_ref, w_ref, d_ref, o0_ref, o1_ref):
        dinv = lax.rsqrt(d_ref[...] + 1.0)
        y = jnp.dot(x_ref[...], w_ref[...],
                    preferred_element_type=jnp.float32) * dinv
        o0_ref[...] = y[:, : fo // 2]
        o1_ref[...] = y[:, fo // 2:]

    return pl.pallas_call(
        body,
        grid=(N // R,),
        in_specs=[_row((R, x.shape[1])), _full(W.shape), _row((R, 1))],
        **_halves_out(fo),
    )(x, W, deg2)


def _tc_layer(z0, z1, b, W, deg2):
    fo = W.shape[1]
    fh = z0.shape[1]

    def body(z0_ref, z1_ref, b_ref, w_ref, d_ref, o0_ref, o1_ref):
        dinv = lax.rsqrt(d_ref[...] + 1.0)
        zc = jnp.concatenate([z0_ref[...], z1_ref[...]], axis=1)
        h = jnp.maximum(zc * dinv + b_ref[...], 0.0)
        y = jnp.dot(h, w_ref[...], preferred_element_type=jnp.float32) * dinv
        o0_ref[...] = y[:, : fo // 2]
        o1_ref[...] = y[:, fo // 2:]

    return pl.pallas_call(
        body,
        grid=(N // R,),
        in_specs=[_row((R, fh)), _row((R, fh)), _full(b.shape),
                  _full(W.shape), _row((R, 1))],
        **_halves_out(fo),
    )(z0, z1, b, W, deg2)


def _tc_final(z0, z1, b, deg2):
    fh = z0.shape[1]

    def body(z0_ref, z1_ref, b_ref, d_ref, o_ref):
        dinv = lax.rsqrt(d_ref[...] + 1.0)
        zc = jnp.concatenate([z0_ref[...], z1_ref[...]], axis=1)
        o_ref[...] = jax.nn.sigmoid(zc * dinv + b_ref[...])

    return pl.pallas_call(
        body,
        grid=(N // R,),
        in_specs=[_row((R, fh)), _row((R, fh)), _full(b.shape), _row((R, 1))],
        out_shape=jax.ShapeDtypeStruct((N, 2 * fh), jnp.float32),
        out_specs=_row((R, 2 * fh)),
    )(z0, z1, b, deg2)


def kernel(x, edge_index, W1, b1, W2, b2, W3, b3):
    src = edge_index[0].astype(jnp.int32)
    dst = edge_index[1].astype(jnp.int32)
    ecnk = jnp.stack([src.reshape(-1, CH), dst.reshape(-1, CH)], axis=1)

    degf = _sc_degree(dst)
    deg2 = (degf[:N] + degf[NPAD:NPAD + N]).reshape(N, 1)

    y0, y1 = _tc_layer1(x, W1, deg2)
    z0, z1 = _sc_propagate(y0, y1, ecnk, 128)

    y0, y1 = _tc_layer(z0, z1, b1.reshape(1, -1), W2, deg2)
    z0, z1 = _sc_propagate(y0, y1, ecnk, 128)

    y0, y1 = _tc_layer(z0, z1, b2.reshape(1, -1), W3, deg2)
    z0, z1 = _sc_propagate(y0, y1, ecnk, 32)

    return _tc_final(z0, z1, b3.reshape(1, -1), deg2)


# speedup vs baseline: 24.5086x; 1.0452x over previous
import functools

import jax
import jax.numpy as jnp
from jax import lax
from jax.experimental import pallas as pl
from jax.experimental.pallas import tpu as pltpu
import jax.experimental.pallas.tpu_sc as plsc

N = 10000
E = 320000
NC, NS, L = 2, 16, 16
NW = NC * NS
NPAD = 10240
R = 400
CH = 96
NBUF = 3
G = 2
CH3 = 96
NBUF3 = 4
G3 = 2


def _sc_degree(dst):
    epw = E // NW
    cpt = NPAD // NS
    mesh = plsc.VectorSubcoreMesh(core_axis_name="c", subcore_axis_name="s")

    @functools.partial(
        pl.kernel,
        out_type=jax.ShapeDtypeStruct((2 * NPAD,), jnp.float32),
        mesh=mesh,
        scratch_types=[
            pltpu.VMEM((epw,), jnp.int32),
            pltpu.VMEM((NPAD,), jnp.float32),
            pltpu.VMEM((NS * cpt,), jnp.float32),
            pltpu.VMEM((cpt,), jnp.float32),
            pltpu.VMEM_SHARED((NS * NPAD,), jnp.float32),
        ],
        compiler_params=pltpu.CompilerParams(needs_layout_passes=False),
    )
    def k(dst_hbm, out_hbm, dst_v, hist_v, red_v, res_v, shared):
        c = lax.axis_index("c")
        s = lax.axis_index("s")
        wid = s * NC + c

        pltpu.sync_copy(dst_hbm.at[pl.ds(wid * epw, epw)], dst_v)

        def zero_body(j, _):
            hist_v[pl.ds(j * L, L)] = jnp.zeros((L,), jnp.float32)
            return 0
        lax.fori_loop(0, NPAD // L, zero_body, 0)

        ones = jnp.full((L,), 1.0, jnp.float32)

        def count_body(j, _):
            idx = dst_v[pl.ds(j * L, L)]
            plsc.addupdate_scatter(hist_v, [idx], ones)
            return 0
        lax.fori_loop(0, epw // L, count_body, 0)

        pltpu.sync_copy(hist_v, shared.at[pl.ds(s * NPAD, NPAD)])
        plsc.subcore_barrier()

        for t in range(NS):
            pltpu.sync_copy(shared.at[pl.ds(t * NPAD + s * cpt, cpt)],
                            red_v.at[pl.ds(t * cpt, cpt)])

        def red_body(j, _):
            acc = jnp.zeros((L,), jnp.float32)

            def add_t(t, a):
                return a + red_v[pl.ds(t * cpt + j * L, L)]
            acc = lax.fori_loop(0, NS, add_t, acc)
            res_v[pl.ds(j * L, L)] = acc
            return 0
        lax.fori_loop(0, cpt // L, red_body, 0)

        pltpu.sync_copy(res_v, out_hbm.at[pl.ds(c * NPAD + s * cpt, cpt)])

    return k(dst)


def _sc_propagate(y0, y1, ecnk, fh, ch, nbuf, g):
    nch = ecnk.shape[0] // NS
    NI = 2 * nbuf
    ngrp2 = nch // NI
    tail = nch - ngrp2 * NI
    rpt = NPAD // NS
    mesh = plsc.VectorSubcoreMesh(core_axis_name="c", subcore_axis_name="s")
    half = jax.ShapeDtypeStruct((NPAD, fh), jnp.float32)

    @functools.partial(
        pl.kernel,
        out_type=(half, half),
        mesh=mesh,
        scratch_types=[
            pltpu.VMEM((NI, 2, ch), jnp.int32),
            pltpu.VMEM((nbuf, ch, fh), jnp.float32),
            pltpu.VMEM_SHARED((NPAD, fh), jnp.float32),
            [pltpu.SemaphoreType.DMA] * NI,
            [pltpu.SemaphoreType.DMA] * nbuf,
            [pltpu.SemaphoreType.DMA] * nbuf,
        ],
        compiler_params=pltpu.CompilerParams(needs_layout_passes=False,
                                             use_tc_tiling_on_sc=False),
    )
    def k(y0_hbm, y1_hbm, e_hbm, z0_hbm, z1_hbm,
          ibuf, gbuf, accum, isems, gsems, ssems):
        c = lax.axis_index("c")
        s = lax.axis_index("s")
        base = s * nch

        rlo = s * rpt

        @pl.when(c == 0)
        def _():
            pltpu.sync_copy(y0_hbm.at[pl.ds(rlo, rpt)],
                            accum.at[pl.ds(rlo, rpt)])

        @pl.when(c == 1)
        def _():
            pltpu.sync_copy(y1_hbm.at[pl.ds(rlo, rpt)],
                            accum.at[pl.ds(rlo, rpt)])

        def fire_idx(j, islot):
            pltpu.async_copy(e_hbm.at[base + j], ibuf.at[islot], isems[islot])

        def wait_idx(islot):
            pltpu.make_async_copy(e_hbm.at[0], ibuf.at[islot],
                                  isems[islot]).wait()

        def fire_gather(islot, slot):
            idx = ibuf.at[islot, 0]

            @pl.when(c == 0)
            def _():
                pltpu.async_copy(y0_hbm.at[idx], gbuf.at[slot], gsems[slot])

            @pl.when(c == 1)
            def _():
                pltpu.async_copy(y1_hbm.at[idx], gbuf.at[slot], gsems[slot])

        def wait_gather(slot):
            pltpu.make_async_copy(y0_hbm.at[pl.ds(0, ch)], gbuf.at[slot],
                                  gsems[slot]).wait()

        def fire_scatter(islot, slot):
            pltpu.async_copy(gbuf.at[slot], accum.at[ibuf.at[islot, 1]],
                             ssems[slot], add=True)

        def drain_scatter(slot):
            pltpu.make_async_copy(gbuf.at[slot], accum.at[ibuf.at[0, 1]],
                                  ssems[slot]).wait()

        plsc.subcore_barrier()

        for b in range(nbuf):
            fire_idx(b, b)
        for b in range(g):
            wait_idx(b)
            fire_gather(b, b)

        def chunk_step(j, b, islot):
            gslot = (b + g) % nbuf
            gislot = (islot + g) % NI

            @pl.when(j < nch - g)
            def _():
                @pl.when(j + g >= nbuf)
                def _():
                    drain_scatter(gslot)
                wait_idx(gislot)
                fire_gather(gislot, gslot)

            wait_gather(b)
            fire_scatter(islot, b)

            @pl.when(j < nch - nbuf)
            def _():
                fire_idx(j + nbuf, (islot + nbuf) % NI)

        def outer(g2, _):
            for p in range(NI):
                j = g2 * NI + p
                chunk_step(j, p % nbuf, p)
            return 0
        lax.fori_loop(0, ngrp2, outer, 0)

        for p in range(tail):
            j = ngrp2 * NI + p
            chunk_step(jnp.int32(j), p % nbuf, p)

        for b in range(nbuf):
            drain_scatter(b)

        plsc.subcore_barrier()

        @pl.when(c == 0)
        def _():
            pltpu.sync_copy(accum.at[pl.ds(rlo, rpt)],
                            z0_hbm.at[pl.ds(rlo, rpt)])

        @pl.when(c == 1)
        def _():
            pltpu.sync_copy(accum.at[pl.ds(rlo, rpt)],
                            z1_hbm.at[pl.ds(rlo, rpt)])

    return k(y0, y1, ecnk)


def _full(shape):
    nd = len(shape)
    return pl.BlockSpec(shape, lambda i: (0,) * nd)


def _row(block):
    return pl.BlockSpec(block, lambda i: (i,) + (0,) * (len(block) - 1))


def _halves_out(fo):
    h = jax.ShapeDtypeStruct((NPAD, fo // 2), jnp.float32)
    return dict(
        out_shape=(h, h),
        out_specs=(_row((R, fo // 2)), _row((R, fo // 2))),
    )


def _tc_layer1(x, W, deg2):
    fo = W.shape[1]

    def body(x_ref, w_ref, d_ref, o0_ref, o1_ref):
        dinv = lax.rsqrt(d_ref[...] + 1.0)
        y = jnp.dot(x_ref[...], w_ref[...],
                    preferred_element_type=jnp.float32) * dinv
        o0_ref[...] = y[:, : fo // 2]
        o1_ref[...] = y[:, fo // 2:]

    return pl.pallas_call(
        body,
        grid=(N // R,),
        in_specs=[_row((R, x.shape[1])), _full(W.shape), _row((R, 1))],
        **_halves_out(fo),
    )(x, W, deg2)


def _tc_layer(z0, z1, b, W, deg2):
    fo = W.shape[1]
    fh = z0.shape[1]

    def body(z0_ref, z1_ref, b_ref, w_ref, d_ref, o0_ref, o1_ref):
        dinv = lax.rsqrt(d_ref[...] + 1.0)
        zc = jnp.concatenate([z0_ref[...], z1_ref[...]], axis=1)
        h = jnp.maximum(zc * dinv + b_ref[...], 0.0)
        y = jnp.dot(h, w_ref[...], preferred_element_type=jnp.float32) * dinv
        o0_ref[...] = y[:, : fo // 2]
        o1_ref[...] = y[:, fo // 2:]

    return pl.pallas_call(
        body,
        grid=(N // R,),
        in_specs=[_row((R, fh)), _row((R, fh)), _full(b.shape),
                  _full(W.shape), _row((R, 1))],
        **_halves_out(fo),
    )(z0, z1, b, W, deg2)


def _tc_final(z0, z1, b, deg2):
    fh = z0.shape[1]

    def body(z0_ref, z1_ref, b_ref, d_ref, o_ref):
        dinv = lax.rsqrt(d_ref[...] + 1.0)
        zc = jnp.concatenate([z0_ref[...], z1_ref[...]], axis=1)
        o_ref[...] = jax.nn.sigmoid(zc * dinv + b_ref[...])

    return pl.pallas_call(
        body,
        grid=(N // R,),
        in_specs=[_row((R, fh)), _row((R, fh)), _full(b.shape), _row((R, 1))],
        out_shape=jax.ShapeDtypeStruct((N, 2 * fh), jnp.float32),
        out_specs=_row((R, 2 * fh)),
    )(z0, z1, b, deg2)


def _chunk_edges(src, dst, ch):
    ept = E // NS
    cpt = -(-ept // ch)
    npe = cpt * ch - ept
    s16 = src.reshape(NS, ept)
    d16 = dst.reshape(NS, ept)
    if npe:
        flat = jnp.arange(NS * npe, dtype=jnp.int32).reshape(NS, npe)
        s16 = jnp.concatenate([s16, flat % N], axis=1)
        d16 = jnp.concatenate([d16, N + flat % (NPAD - N)], axis=1)
    s16 = s16.reshape(NS, cpt, ch)
    d16 = d16.reshape(NS, cpt, ch)
    return jnp.stack([s16, d16], axis=2).reshape(NS * cpt, 2, ch)


def kernel(x, edge_index, W1, b1, W2, b2, W3, b3):
    src = edge_index[0].astype(jnp.int32)
    dst = edge_index[1].astype(jnp.int32)
    ecnk = _chunk_edges(src, dst, CH)
    ecnk3 = _chunk_edges(src, dst, CH3)

    degf = _sc_degree(dst)
    deg2 = (degf[:N] + degf[NPAD:NPAD + N]).reshape(N, 1)

    y0, y1 = _tc_layer1(x, W1, deg2)
    z0, z1 = _sc_propagate(y0, y1, ecnk, 128, CH, NBUF, G)

    y0, y1 = _tc_layer(z0, z1, b1.reshape(1, -1), W2, deg2)
    z0, z1 = _sc_propagate(y0, y1, ecnk, 128, CH, NBUF, G)

    y0, y1 = _tc_layer(z0, z1, b2.reshape(1, -1), W3, deg2)
    z0, z1 = _sc_propagate(y0, y1, ecnk3, 32, CH3, NBUF3, G3)

    return _tc_final(z0, z1, b3.reshape(1, -1), deg2)
